# baseline (device time: 8069 ns/iter reference)
import jax
import jax.numpy as jnp
from jax import lax
from jax.experimental import pallas as pl
from jax.experimental.pallas import tpu as pltpu

N_GLOBAL_COLS = 1024
EPS = 1e-5
LANES = 128


def kernel(x, gamma):
    m, n = x.shape
    mh = m // 2
    nblk = mh // LANES

    def body(
        x_ref,
        g_ref,
        out_ref,
        packed_a,
        packed_b,
        recv_a,
        recv_b,
        send_sems,
        recv_sems,
    ):
        my_x = lax.axis_index("x")
        my_y = lax.axis_index("y")
        nbr = (my_x, 1 - my_y)

        barrier_sem = pltpu.get_barrier_semaphore()
        pl.semaphore_signal(
            barrier_sem, inc=1, device_id=nbr,
            device_id_type=pl.DeviceIdType.MESH,
        )

        f32 = jnp.float32
        Lh = (
            lax.broadcasted_iota(jnp.int32, (mh, LANES), 0) % LANES
            == lax.broadcasted_iota(jnp.int32, (mh, LANES), 1)
        ).astype(f32)
        Rth = (
            lax.broadcasted_iota(jnp.int32, (nblk, mh), 1) // LANES
            == lax.broadcasted_iota(jnp.int32, (nblk, mh), 0)
        ).astype(f32)

        def make_rdma(src, dst, slot):
            return pltpu.make_async_remote_copy(
                src_ref=src,
                dst_ref=dst,
                send_sem=send_sems.at[slot],
                recv_sem=recv_sems.at[slot],
                device_id=nbr,
                device_id_type=pl.DeviceIdType.MESH,
            )

        xa = x_ref[0:mh, :]
        ssq_a = jnp.sum(xa * xa, axis=1, keepdims=True)
        packed_a[:, :] = jnp.dot(Rth, Lh * ssq_a, preferred_element_type=f32)
        pl.semaphore_wait(barrier_sem, 1)
        rdma_a = make_rdma(packed_a, recv_a, 0)
        rdma_a.start()

        xb = x_ref[mh:m, :]
        ssq_b = jnp.sum(xb * xb, axis=1, keepdims=True)
        packed_b[:, :] = jnp.dot(Rth, Lh * ssq_b, preferred_element_type=f32)
        rdma_b = make_rdma(packed_b, recv_b, 1)
        rdma_b.start()

        g = g_ref[:]
        xga = xa * g
        xgb = xb * g
        Rh = (
            lax.broadcasted_iota(jnp.int32, (mh, nblk), 0) // LANES
            == lax.broadcasted_iota(jnp.int32, (mh, nblk), 1)
        ).astype(f32)

        def unpack_inv(packed, recv):
            total = packed[:, :] + recv[:, :]
            v = jnp.sum(
                jnp.dot(Rh, total, preferred_element_type=f32) * Lh,
                axis=1,
                keepdims=True,
            )
            return lax.rsqrt(v * (1.0 / N_GLOBAL_COLS) + EPS)

        rdma_a.wait()
        out_ref[0:mh, :] = xga * unpack_inv(packed_a, recv_a)
        rdma_b.wait()
        out_ref[mh:m, :] = xgb * unpack_inv(packed_b, recv_b)

    return pl.pallas_call(
        body,
        out_shape=jax.ShapeDtypeStruct((m, n), x.dtype),
        in_specs=[
            pl.BlockSpec(memory_space=pltpu.VMEM),
            pl.BlockSpec(memory_space=pltpu.VMEM),
        ],
        out_specs=pl.BlockSpec(memory_space=pltpu.VMEM),
        scratch_shapes=[
            pltpu.VMEM((nblk, LANES), jnp.float32),
            pltpu.VMEM((nblk, LANES), jnp.float32),
            pltpu.VMEM((nblk, LANES), jnp.float32),
            pltpu.VMEM((nblk, LANES), jnp.float32),
            pltpu.SemaphoreType.DMA((2,)),
            pltpu.SemaphoreType.DMA((2,)),
        ],
        compiler_params=pltpu.CompilerParams(collective_id=0),
    )(x, gamma)


# device time: 7895 ns/iter; 1.0220x vs baseline; 1.0220x over previous
import jax
import jax.numpy as jnp
from jax import lax
from jax.experimental import pallas as pl
from jax.experimental.pallas import tpu as pltpu

N_GLOBAL_COLS = 1024
EPS = 1e-5
LANES = 128


def kernel(x, gamma):
    m, n = x.shape
    nblk = m // LANES

    def body(x_ref, g_ref, out_ref, packed_ref, recv_ref, send_sem, recv_sem):
        my_x = lax.axis_index("x")
        my_y = lax.axis_index("y")
        nbr = (my_x, 1 - my_y)

        barrier_sem = pltpu.get_barrier_semaphore()
        pl.semaphore_signal(
            barrier_sem, inc=1, device_id=nbr,
            device_id_type=pl.DeviceIdType.MESH,
        )

        xv = x_ref[:, :]
        ssq = jnp.sum(xv * xv, axis=1, keepdims=True)

        f32 = jnp.float32
        L = (
            lax.broadcasted_iota(jnp.int32, (m, LANES), 0) % LANES
            == lax.broadcasted_iota(jnp.int32, (m, LANES), 1)
        ).astype(f32)
        Rt = (
            lax.broadcasted_iota(jnp.int32, (nblk, m), 1) // LANES
            == lax.broadcasted_iota(jnp.int32, (nblk, m), 0)
        ).astype(f32)

        packed_ref[:, :] = jnp.dot(Rt, L * ssq, preferred_element_type=f32)

        pl.semaphore_wait(barrier_sem, 1)

        rdma = pltpu.make_async_remote_copy(
            src_ref=packed_ref,
            dst_ref=recv_ref,
            send_sem=send_sem,
            recv_sem=recv_sem,
            device_id=nbr,
            device_id_type=pl.DeviceIdType.MESH,
        )
        rdma.start()
        xg = xv * g_ref[:]
        R = (
            lax.broadcasted_iota(jnp.int32, (m, nblk), 0) // LANES
            == lax.broadcasted_iota(jnp.int32, (m, nblk), 1)
        ).astype(f32)
        rdma.wait()

        total = packed_ref[:, :] + recv_ref[:, :]
        v = jnp.sum(
            jnp.dot(R, total, preferred_element_type=f32) * L,
            axis=1,
            keepdims=True,
        )
        inv_rms = lax.rsqrt(v * (1.0 / N_GLOBAL_COLS) + EPS)
        out_ref[:, :] = xg * inv_rms

    return pl.pallas_call(
        body,
        out_shape=jax.ShapeDtypeStruct((m, n), x.dtype),
        in_specs=[
            pl.BlockSpec(memory_space=pltpu.VMEM),
            pl.BlockSpec(memory_space=pltpu.VMEM),
        ],
        out_specs=pl.BlockSpec(memory_space=pltpu.VMEM),
        scratch_shapes=[
            pltpu.VMEM((nblk, LANES), jnp.float32),
            pltpu.VMEM((nblk, LANES), jnp.float32),
            pltpu.SemaphoreType.DMA,
            pltpu.SemaphoreType.DMA,
        ],
        compiler_params=pltpu.CompilerParams(collective_id=0),
    )(x, gamma)
